# baseline (speedup 1.0000x reference)
import functools

import jax
import jax.numpy as jnp
from jax import lax
from jax.experimental import pallas as pl
from jax.experimental.pallas import tpu as pltpu
from jax.experimental.pallas import tpu_sc as plsc

_NUM_EXPERTS = 64
_MULT_MOD = 21
_SEED_MOD = 42
_LANES = 16
_NUM_WORKERS = 32


def _sc_fill(n_flat: int):
    chunk = n_flat // _NUM_WORKERS
    steps = chunk // _LANES
    mesh = plsc.VectorSubcoreMesh(core_axis_name="c", subcore_axis_name="s")

    @functools.partial(
        pl.kernel,
        mesh=mesh,
        out_type=jax.ShapeDtypeStruct((n_flat,), jnp.int32),
        scratch_types=[pltpu.VMEM((chunk,), jnp.int32)],
    )
    def fill(out_hbm, buf):
        i32 = lambda v: jnp.int32(v)
        wid = lax.axis_index("s") * i32(2) + lax.axis_index("c")
        base = wid * i32(chunk)
        lane = lax.iota(jnp.int32, _LANES)
        cvec = (
            i32(_MULT_MOD) * (lane >> i32(2))
            + i32(_SEED_MOD)
            + ((lane >> i32(1)) & i32(1))
        )
        mvec = ((lane & i32(1)) - i32(1)) & i32(_NUM_EXPERTS - 1)
        sbase = i32(_MULT_MOD) * (base >> i32(2))

        def body(j, carry):
            off, s = carry
            buf[pl.ds(off, _LANES)] = (cvec + s) & mvec
            return (off + i32(_LANES), s + i32(_MULT_MOD * 4))

        lax.fori_loop(0, steps, body, (i32(0), sbase))
        pltpu.sync_copy(buf, out_hbm.at[pl.ds(base, chunk)])

    return fill


def kernel(x):
    batch, seq, _ = x.shape
    n = batch * seq
    out32 = _sc_fill(4 * n)()
    return lax.bitcast_convert_type(out32.reshape(n, 2, 2), jnp.int64)
